# baseline (speedup 1.0000x reference)
import functools

import jax
import jax.numpy as jnp
from jax import lax
from jax.experimental import pallas as pl
from jax.experimental.pallas import tpu as pltpu
from jax.experimental.pallas import tpu_sc as plsc


def _reduce0(a, op, keep=8):
    while a.shape[0] > keep:
        h = a.shape[0] // 2
        a = op(a[:h], a[h:])
    return a


def _dist_keys_body(x_ref, keys_ref, flat_ref):
    xb = x_ref[0]
    s = xb.shape[0]
    inf = jnp.float32(jnp.inf)

    g = jax.lax.dot_general(
        xb, xb, (((1,), (1,)), ((), ())),
        preferred_element_type=jnp.float32,
    )
    n = jnp.sum(xb * xb, axis=1)
    d2 = jnp.maximum(n[:, None] + n[None, :] - 2.0 * g, 0.0)
    d = jnp.sqrt(d2)

    sum_n = jnp.sum(n)
    col_g = jnp.sum(_reduce0(g, jnp.add), axis=0, keepdims=True)
    sum_d = jnp.sum(_reduce0(d, jnp.add), axis=0, keepdims=True)
    sum_d2 = sum_n + jnp.float32(s) * n[None, :] - 2.0 * col_g
    rvar = (sum_d2 - sum_d * sum_d * (1.0 / s)) * (1.0 / (s - 1))
    flat = 1.0 / (rvar + 1e-6)
    flat_ref[0] = 1.0 / (1.0 + jnp.exp(flat * -0.1))

    row = jax.lax.broadcasted_iota(jnp.int32, (s, s), 0)
    col = jax.lax.broadcasted_iota(jnp.int32, (s, s), 1)
    keys_ref[0] = jnp.where(
        row == col,
        inf,
        jax.lax.bitcast_convert_type(
            (jax.lax.bitcast_convert_type(d, jnp.int32) & ~jnp.int32(0x1FF))
            | col,
            jnp.float32,
        ),
    )


def _make_sc_topk(r_total, s, rows_per_w):
    mesh = plsc.VectorSubcoreMesh(core_axis_name="c", subcore_axis_name="s")

    @functools.partial(
        pl.kernel,
        mesh=mesh,
        out_type=[jax.ShapeDtypeStruct((r_total,), jnp.float32)] * 2,
        scratch_types=[
            pltpu.VMEM((rows_per_w, s), jnp.float32),
            pltpu.VMEM((rows_per_w,), jnp.float32),
            pltpu.VMEM((rows_per_w,), jnp.float32),
        ],
    )
    def sc_topk(keys_hbm, tree_hbm, cyc_hbm, buf, tbuf, cbuf):
        wid = lax.axis_index("s") * 2 + lax.axis_index("c")
        base = wid * rows_per_w
        pltpu.sync_copy(keys_hbm.at[pl.ds(base, rows_per_w)], buf)
        nchunk = s // 16
        inf = jnp.float32(jnp.inf)

        lanes = lax.iota(jnp.int32, 16)

        dnums = lax.GatherDimensionNumbers(
            offset_dims=(), collapsed_slice_dims=(0,), start_index_map=(0,))

        def shuffle(v, idx):
            return lax.gather(
                v, idx[:, None], dimension_numbers=dnums, slice_sizes=(1,),
                mode=lax.GatherScatterMode.PROMISE_IN_BOUNDS)

        def all_min(v):
            for off in (8, 4, 2, 1):
                v = jnp.minimum(v, shuffle(v, lanes ^ off))
            return v

        def per_group(grp, _):
            def per_row(r2, carry):
                tvec, cvec = carry
                r = grp * 16 + r2
                ms = []
                for rnd in range(4):
                    def chunk_min(c, acc):
                        return jnp.minimum(acc, buf[r, pl.ds(c * 16, 16)])

                    acc = lax.fori_loop(0, nchunk, chunk_min,
                                        jnp.full((16,), inf, jnp.float32))
                    mk = all_min(acc)
                    ms.append(mk)
                    if rnd < 3:
                        def chunk_mask(c, _c):
                            v = buf[r, pl.ds(c * 16, 16)]
                            buf[r, pl.ds(c * 16, 16)] = jnp.where(
                                v == mk, inf, v)
                            return 0

                        lax.fori_loop(0, nchunk, chunk_mask, 0)

                m1, m2, m3, m4 = (
                    lax.bitcast_convert_type(
                        lax.bitcast_convert_type(mk, jnp.int32)
                        & ~jnp.int32(0x1FF),
                        jnp.float32,
                    )
                    for mk in ms
                )
                tree = m4 / jnp.maximum(m1, 1e-6)
                nmean = (m1 + m2 + m3 + m4) * 0.2
                e1, e2, e3, e4 = m1 - nmean, m2 - nmean, m3 - nmean, m4 - nmean
                nvar = (
                    nmean * nmean + e1 * e1 + e2 * e2 + e3 * e3 + e4 * e4
                ) * 0.25
                cyc = 1.0 / (nvar + 1e-6)
                sel = lanes == r2
                return (jnp.where(sel, tree, tvec),
                        jnp.where(sel, cyc, cvec))

            z = jnp.zeros((16,), jnp.float32)
            tvec, cvec = lax.fori_loop(0, 16, per_row, (z, z))
            tbuf[pl.ds(grp * 16, 16)] = 1.0 / (1.0 + jnp.exp(tvec * -0.1))
            cbuf[pl.ds(grp * 16, 16)] = 1.0 / (1.0 + jnp.exp(cvec * -0.1))
            return 0

        lax.fori_loop(0, rows_per_w // 16, per_group, 0)
        pltpu.sync_copy(tbuf, tree_hbm.at[pl.ds(base, rows_per_w)])
        pltpu.sync_copy(cbuf, cyc_hbm.at[pl.ds(base, rows_per_w)])

    return sc_topk


def kernel(x):
    b, s, dmodel = x.shape
    keys, flat = pl.pallas_call(
        _dist_keys_body,
        grid=(b,),
        in_specs=[pl.BlockSpec((1, s, dmodel), lambda i: (i, 0, 0))],
        out_specs=[
            pl.BlockSpec((1, s, s), lambda i: (i, 0, 0)),
            pl.BlockSpec((1, 1, s), lambda i: (i, 0, 0)),
        ],
        out_shape=[
            jax.ShapeDtypeStruct((b, s, s), jnp.float32),
            jax.ShapeDtypeStruct((b, 1, s), jnp.float32),
        ],
    )(x)
    r_total = b * s
    tree, cyc = _make_sc_topk(r_total, s, r_total // 32)(
        keys.reshape(r_total, s))
    return jnp.stack(
        [tree.reshape(b, s), cyc.reshape(b, s), flat[:, 0, :]], axis=-1)


# speedup vs baseline: 8.4487x; 8.4487x over previous
import jax
import jax.numpy as jnp
from jax.experimental import pallas as pl


def _reduce0(a, op, keep=8):
    while a.shape[0] > keep:
        h = a.shape[0] // 2
        a = op(a[:h], a[h:])
    return a


def _features_body(x_ref, tree_ref, cyc_ref, flat_ref):
    xb = x_ref[0]
    s = xb.shape[0]
    inf = jnp.float32(jnp.inf)

    g = jax.lax.dot_general(
        xb, xb, (((1,), (1,)), ((), ())),
        preferred_element_type=jnp.float32,
    )
    n = jnp.sum(xb * xb, axis=1)
    d2 = jnp.maximum(n[:, None] + n[None, :] - 2.0 * g, 0.0)
    d = jnp.sqrt(d2)

    sum_n = jnp.sum(n)
    col_g = jnp.sum(_reduce0(g, jnp.add), axis=0, keepdims=True)
    sum_d = jnp.sum(_reduce0(d, jnp.add), axis=0, keepdims=True)
    sum_d2 = sum_n + jnp.float32(s) * n[None, :] - 2.0 * col_g
    rvar = (sum_d2 - sum_d * sum_d * (1.0 / s)) * (1.0 / (s - 1))

    row = jax.lax.broadcasted_iota(jnp.int32, (s, s), 0)
    col = jax.lax.broadcasted_iota(jnp.int32, (s, s), 1)
    key = jnp.where(
        row == col,
        inf,
        jax.lax.bitcast_convert_type(
            (jax.lax.bitcast_convert_type(d2, jnp.int32) & ~jnp.int32(0x1FF))
            | row,
            jnp.float32,
        ),
    )
    ms = []
    for r in range(4):
        mk = jnp.min(_reduce0(key, jnp.minimum), axis=0, keepdims=True)
        ms.append(mk)
        if r < 3:
            key = jnp.where(key == mk, inf, key)

    m1, m2, m3, m4 = (
        jnp.sqrt(jax.lax.bitcast_convert_type(
            jax.lax.bitcast_convert_type(mk, jnp.int32) & ~jnp.int32(0x1FF),
            jnp.float32,
        ))
        for mk in ms
    )
    tree = m4 / jnp.maximum(m1, 1e-6)
    nmean = (m1 + m2 + m3 + m4) * 0.2
    nvar = (
        nmean * nmean
        + (m1 - nmean) ** 2 + (m2 - nmean) ** 2
        + (m3 - nmean) ** 2 + (m4 - nmean) ** 2
    ) * 0.25
    cyc = 1.0 / (nvar + 1e-6)
    flat = 1.0 / (rvar + 1e-6)

    stacked = jnp.concatenate([tree, cyc, flat], axis=0)
    feats = 1.0 / (1.0 + jnp.exp(stacked * -0.1))
    tree_ref[0] = feats[0:1]
    cyc_ref[0] = feats[1:2]
    flat_ref[0] = feats[2:3]


def kernel(x):
    b, s, dmodel = x.shape
    out = jax.ShapeDtypeStruct((b, 1, s), jnp.float32)
    tree, cyc, flat = pl.pallas_call(
        _features_body,
        grid=(b,),
        in_specs=[pl.BlockSpec((1, s, dmodel), lambda i: (i, 0, 0))],
        out_specs=[pl.BlockSpec((1, 1, s), lambda i: (i, 0, 0))] * 3,
        out_shape=[out] * 3,
    )(x)
    return jnp.concatenate([tree, cyc, flat], axis=1).transpose(0, 2, 1)


# speedup vs baseline: 8.7669x; 1.0377x over previous
import jax
import jax.numpy as jnp
from jax.experimental import pallas as pl


def _reduce0(a, op, keep=8):
    while a.shape[0] > keep:
        h = a.shape[0] // 2
        a = op(a[:h], a[h:])
    return a


def _features_body(x_ref, tree_ref, cyc_ref, flat_ref):
    xb = x_ref[0]
    s = xb.shape[0]
    inf = jnp.float32(jnp.inf)

    g = jax.lax.dot_general(
        xb, xb, (((1,), (1,)), ((), ())),
        preferred_element_type=jnp.float32,
    )
    n = jnp.sum(xb * xb, axis=1)
    d2 = jnp.maximum(n[:, None] + n[None, :] - 2.0 * g, 0.0)
    d = d2 * jax.lax.rsqrt(d2 + 1e-20)

    sum_n = jnp.sum(n)
    col_g = jnp.sum(_reduce0(g, jnp.add), axis=0, keepdims=True)
    sum_d = jnp.sum(_reduce0(d, jnp.add), axis=0, keepdims=True)
    sum_d2 = sum_n + jnp.float32(s) * n[None, :] - 2.0 * col_g
    rvar = (sum_d2 - sum_d * sum_d * (1.0 / s)) * (1.0 / (s - 1))

    row = jax.lax.broadcasted_iota(jnp.int32, (s, s), 0)
    col = jax.lax.broadcasted_iota(jnp.int32, (s, s), 1)
    key = jnp.where(
        row == col,
        inf,
        jax.lax.bitcast_convert_type(
            (jax.lax.bitcast_convert_type(d2, jnp.int32) & ~jnp.int32(0x1FF))
            | row,
            jnp.float32,
        ),
    )
    ms = []
    for r in range(4):
        mk = jnp.min(_reduce0(key, jnp.minimum), axis=0, keepdims=True)
        ms.append(mk)
        if r < 3:
            key = jnp.where(key == mk, inf, key)

    m1, m2, m3, m4 = (
        jnp.sqrt(jax.lax.bitcast_convert_type(
            jax.lax.bitcast_convert_type(mk, jnp.int32) & ~jnp.int32(0x1FF),
            jnp.float32,
        ))
        for mk in ms
    )
    tree = m4 / jnp.maximum(m1, 1e-6)
    nmean = (m1 + m2 + m3 + m4) * 0.2
    nvar = (
        nmean * nmean
        + (m1 - nmean) ** 2 + (m2 - nmean) ** 2
        + (m3 - nmean) ** 2 + (m4 - nmean) ** 2
    ) * 0.25
    cyc = 1.0 / (nvar + 1e-6)
    flat = 1.0 / (rvar + 1e-6)

    stacked = jnp.concatenate([tree, cyc, flat], axis=0)
    feats = 1.0 / (1.0 + jnp.exp(stacked * -0.1))
    tree_ref[0] = feats[0:1]
    cyc_ref[0] = feats[1:2]
    flat_ref[0] = feats[2:3]


def kernel(x):
    b, s, dmodel = x.shape
    out = jax.ShapeDtypeStruct((b, 1, s), jnp.float32)
    tree, cyc, flat = pl.pallas_call(
        _features_body,
        grid=(b,),
        in_specs=[pl.BlockSpec((1, s, dmodel), lambda i: (i, 0, 0))],
        out_specs=[pl.BlockSpec((1, 1, s), lambda i: (i, 0, 0))] * 3,
        out_shape=[out] * 3,
    )(x)
    return jnp.concatenate([tree, cyc, flat], axis=1).transpose(0, 2, 1)
